# baseline (speedup 1.0000x reference)
import functools

import jax
import jax.numpy as jnp
from jax import lax
from jax.experimental import pallas as pl
from jax.experimental.pallas import tpu as pltpu
from jax.experimental.pallas import tpu_sc as plsc

VOCAB = 100000
D = 128
B = 4096
H = 50
N = B * H
NW = 32
PER_W = N // NW
CH = 128
NCH = PER_W // CH
NBUF = 6
MAX_NORM = 1.0
EPS = 1e-7


@functools.partial(
    pl.kernel,
    out_type=jax.ShapeDtypeStruct((N, D), jnp.float32),
    mesh=plsc.VectorSubcoreMesh(core_axis_name="c", subcore_axis_name="s"),
    scratch_types=[
        pltpu.VMEM((NCH, CH), jnp.int32),
        pltpu.VMEM((NBUF, CH, D), jnp.float32),
        pltpu.SemaphoreType.DMA((NBUF,)),
        pltpu.SemaphoreType.DMA((NBUF,)),
    ],
)
def _emb_lookup(x_hbm, table_hbm, out_hbm, idx_all, rows, gsem, osem):
    cid = lax.axis_index("c")
    sid = lax.axis_index("s")
    wid = sid * 2 + cid
    base = wid * PER_W

    pltpu.sync_copy(x_hbm.at[wid], idx_all)

    for g0 in range(NBUF - 1):
        pltpu.async_copy(table_hbm.at[idx_all.at[g0]], rows.at[g0],
                         gsem.at[g0])

    lanes = lax.iota(jnp.int32, 16)
    perms = [lanes ^ (1 << k) for k in range(4)]

    def block_body(g, carry):
        b = lax.rem(g, NBUF)

        pltpu.make_async_copy(
            table_hbm.at[idx_all.at[g]], rows.at[b], gsem.at[b]).wait()

        @pl.when(g + NBUF - 1 < NCH)
        def _refill():
            bn = lax.rem(g + NBUF - 1, NBUF)

            @pl.when(g >= 1)
            def _wait_store():
                pltpu.make_async_copy(
                    rows.at[bn], out_hbm.at[pl.ds(0, CH)],
                    osem.at[bn]).wait()

            pltpu.async_copy(table_hbm.at[idx_all.at[g + NBUF - 1]],
                             rows.at[bn], gsem.at[bn])

        rows_b = rows.at[b]

        def row_body(r, c2):
            vs = [rows_b[r, pl.ds(16 * j, 16)] for j in range(8)]
            a = [jnp.abs(v) for v in vs]
            s01 = a[0] + a[1]
            s23 = a[2] + a[3]
            s45 = a[4] + a[5]
            s67 = a[6] + a[7]
            acc = (s01 + s23) + (s45 + s67)
            for p in perms:
                acc = acc + acc.at[p].get(mode="promise_in_bounds",
                                          unique_indices=True)
            scale = jnp.minimum(MAX_NORM / (acc + EPS), jnp.float32(1.0))
            for j in range(8):
                rows_b[r, pl.ds(16 * j, 16)] = vs[j] * scale
            return c2

        lax.fori_loop(0, CH, row_body, 0, unroll=8)

        pltpu.async_copy(rows_b, out_hbm.at[pl.ds(base + g * CH, CH)],
                         osem.at[b])

        return carry

    lax.fori_loop(0, NCH, block_body, 0)

    for k in range(NCH - NBUF, NCH):
        pltpu.make_async_copy(
            rows.at[k % NBUF], out_hbm.at[pl.ds(0, CH)],
            osem.at[k % NBUF]).wait()


def kernel(x, table):
    xt = x.T.astype(jnp.int32).reshape(NW, NCH, CH)
    out = _emb_lookup(xt, table)
    return out.reshape(H, B, D).swapaxes(0, 1)


# speedup vs baseline: 1.0026x; 1.0026x over previous
import functools

import jax
import jax.numpy as jnp
from jax import lax
from jax.experimental import pallas as pl
from jax.experimental.pallas import tpu as pltpu
from jax.experimental.pallas import tpu_sc as plsc

VOCAB = 100000
D = 128
B = 4096
H = 50
N = B * H
NW = 32
PER_W = N // NW
CH = 128
NCH = PER_W // CH
NBUF = 7
MAX_NORM = 1.0
EPS = 1e-7


@functools.partial(
    pl.kernel,
    out_type=jax.ShapeDtypeStruct((N, D), jnp.float32),
    mesh=plsc.VectorSubcoreMesh(core_axis_name="c", subcore_axis_name="s"),
    scratch_types=[
        pltpu.VMEM((NCH, CH), jnp.int32),
        pltpu.VMEM((NBUF, CH, D), jnp.float32),
        pltpu.SemaphoreType.DMA((NBUF,)),
        pltpu.SemaphoreType.DMA((NBUF,)),
    ],
)
def _emb_lookup(x_hbm, table_hbm, out_hbm, idx_all, rows, gsem, osem):
    cid = lax.axis_index("c")
    sid = lax.axis_index("s")
    wid = sid * 2 + cid
    base = wid * PER_W

    pltpu.sync_copy(x_hbm.at[wid], idx_all)

    for g0 in range(NBUF - 1):
        pltpu.async_copy(table_hbm.at[idx_all.at[g0]], rows.at[g0],
                         gsem.at[g0])

    lanes = lax.iota(jnp.int32, 16)
    perms = [lanes ^ (1 << k) for k in range(4)]

    def block_body(g, carry):
        b = lax.rem(g, NBUF)

        pltpu.make_async_copy(
            table_hbm.at[idx_all.at[g]], rows.at[b], gsem.at[b]).wait()

        @pl.when(g + NBUF - 1 < NCH)
        def _refill():
            bn = lax.rem(g + NBUF - 1, NBUF)

            @pl.when(g >= 1)
            def _wait_store():
                pltpu.make_async_copy(
                    rows.at[bn], out_hbm.at[pl.ds(0, CH)],
                    osem.at[bn]).wait()

            pltpu.async_copy(table_hbm.at[idx_all.at[g + NBUF - 1]],
                             rows.at[bn], gsem.at[bn])

        rows_b = rows.at[b]

        def row_body(r, c2):
            vs = [rows_b[r, pl.ds(16 * j, 16)] for j in range(8)]
            a = [jnp.abs(v) for v in vs]
            s01 = a[0] + a[1]
            s23 = a[2] + a[3]
            s45 = a[4] + a[5]
            s67 = a[6] + a[7]
            acc = (s01 + s23) + (s45 + s67)
            for p in perms:
                acc = acc + acc.at[p].get(mode="promise_in_bounds",
                                          unique_indices=True)
            scale = jnp.minimum(MAX_NORM / (acc + EPS), jnp.float32(1.0))
            for j in range(8):
                rows_b[r, pl.ds(16 * j, 16)] = vs[j] * scale
            return c2

        lax.fori_loop(0, CH, row_body, 0, unroll=8)

        pltpu.async_copy(rows_b, out_hbm.at[pl.ds(base + g * CH, CH)],
                         osem.at[b])

        return carry

    lax.fori_loop(0, NCH, block_body, 0)

    for k in range(NCH - NBUF, NCH):
        pltpu.make_async_copy(
            rows.at[k % NBUF], out_hbm.at[pl.ds(0, CH)],
            osem.at[k % NBUF]).wait()


def kernel(x, table):
    xt = x.T.astype(jnp.int32).reshape(NW, NCH, CH)
    out = _emb_lookup(xt, table)
    return out.reshape(H, B, D).swapaxes(0, 1)
